# baseline (speedup 1.0000x reference)
import jax
import jax.numpy as jnp
from jax import lax
from jax.experimental import pallas as pl
from jax.experimental.pallas import tpu as pltpu
from jax.experimental.pallas import tpu_sc as plsc

N_NODES = 10000
N_EDGES = 320000
D_FEAT = 128

NC = 2
NS = 16
NW = NC * NS

CH = 125
E_ROWS = N_EDGES // CH
RPW = E_ROWS // NW
N_PAD = 10240
ROWS_PER_SUB = N_PAD // NS
CNT_L = 128


def _sc_sums(emb_hbm, col2d, row2d, zacc_hbm, sums_out,
             cidx_v, ridx_v, rows_v, acc_sh, sem):
    cid = lax.axis_index("c")
    sid = lax.axis_index("s")
    wid = cid * NS + sid

    nb = sid * ROWS_PER_SUB
    pltpu.sync_copy(zacc_hbm, acc_sh.at[pl.ds(nb, ROWS_PER_SUB)])
    eb = wid * RPW
    pltpu.sync_copy(col2d.at[pl.ds(eb, RPW)], cidx_v)
    pltpu.sync_copy(row2d.at[pl.ds(eb, RPW)], ridx_v)
    plsc.subcore_barrier()

    def step(j, carry):
        pltpu.async_copy(emb_hbm.at[cidx_v.at[j]], rows_v, sem).wait()
        pltpu.sync_copy(rows_v, acc_sh.at[ridx_v.at[j]], add=True)
        return carry

    lax.fori_loop(0, RPW, step, 0)
    plsc.subcore_barrier()

    pltpu.sync_copy(acc_sh.at[pl.ds(nb, ROWS_PER_SUB)],
                    sums_out.at[cid, pl.ds(nb, ROWS_PER_SUB)])


def _sc_counts(row2d, zcnt_hbm, ones_hbm, cnts_out,
               ridx_v, ones_v, cnt_sh, sem):
    cid = lax.axis_index("c")
    sid = lax.axis_index("s")
    wid = cid * NS + sid

    nb = sid * ROWS_PER_SUB
    pltpu.sync_copy(zcnt_hbm, cnt_sh.at[pl.ds(nb, ROWS_PER_SUB)])
    eb = wid * RPW
    pltpu.sync_copy(row2d.at[pl.ds(eb, RPW)], ridx_v)
    pltpu.sync_copy(ones_hbm, ones_v)
    plsc.subcore_barrier()

    def step(j, carry):
        pltpu.sync_copy(ones_v, cnt_sh.at[ridx_v.at[j]], add=True)
        return carry

    lax.fori_loop(0, RPW, step, 0)
    plsc.subcore_barrier()

    pltpu.sync_copy(cnt_sh.at[pl.ds(nb, ROWS_PER_SUB)],
                    cnts_out.at[cid, pl.ds(nb, ROWS_PER_SUB)])


def _combine_body(sums_ref, cnts_ref, self_ref, out_ref):
    s = sums_ref[0] + sums_ref[1]
    c = cnts_ref[0, :, 0:1] + cnts_ref[1, :, 0:1]
    f1 = s / jnp.maximum(c, 1.0)
    out_ref[:, :D_FEAT] = f1
    out_ref[:, D_FEAT:] = self_ref[:] - f1


def kernel(embedding, edge_index, self_feats):
    row = edge_index[0].astype(jnp.int32).reshape(E_ROWS, CH)
    col = edge_index[1].astype(jnp.int32).reshape(E_ROWS, CH)
    zacc = jnp.zeros((ROWS_PER_SUB, D_FEAT), jnp.float32)
    zcnt = jnp.zeros((ROWS_PER_SUB, CNT_L), jnp.float32)
    ones = jnp.ones((CH, CNT_L), jnp.float32)

    mesh = plsc.VectorSubcoreMesh(core_axis_name="c", subcore_axis_name="s",
                                  num_cores=NC, num_subcores=NS)
    sums_p = pl.kernel(
        _sc_sums,
        out_type=jax.ShapeDtypeStruct((NC, N_PAD, D_FEAT), jnp.float32),
        mesh=mesh,
        scratch_types=[
            pltpu.VMEM((RPW, CH), jnp.int32),
            pltpu.VMEM((RPW, CH), jnp.int32),
            pltpu.VMEM((CH, D_FEAT), jnp.float32),
            pltpu.VMEM_SHARED((N_PAD, D_FEAT), jnp.float32),
            pltpu.SemaphoreType.DMA,
        ],
    )(embedding, col, row, zacc)

    cnts_p = pl.kernel(
        _sc_counts,
        out_type=jax.ShapeDtypeStruct((NC, N_PAD, CNT_L), jnp.float32),
        mesh=mesh,
        scratch_types=[
            pltpu.VMEM((RPW, CH), jnp.int32),
            pltpu.VMEM((CH, CNT_L), jnp.float32),
            pltpu.VMEM_SHARED((N_PAD, CNT_L), jnp.float32),
            pltpu.SemaphoreType.DMA,
        ],
    )(row, zcnt, ones)

    blk = 2000
    out = pl.pallas_call(
        _combine_body,
        grid=(N_NODES // blk,),
        in_specs=[
            pl.BlockSpec((NC, blk, D_FEAT), lambda i: (0, i, 0)),
            pl.BlockSpec((NC, blk, CNT_L), lambda i: (0, i, 0)),
            pl.BlockSpec((blk, D_FEAT), lambda i: (i, 0)),
        ],
        out_specs=pl.BlockSpec((blk, 2 * D_FEAT), lambda i: (i, 0)),
        out_shape=jax.ShapeDtypeStruct((N_NODES, 2 * D_FEAT), jnp.float32),
    )(sums_p, cnts_p, self_feats)
    return out


# speedup vs baseline: 1.0062x; 1.0062x over previous
import jax
import jax.numpy as jnp
from jax import lax
from jax.experimental import pallas as pl
from jax.experimental.pallas import tpu as pltpu
from jax.experimental.pallas import tpu_sc as plsc

N_NODES = 10000
N_EDGES = 320000
D_FEAT = 128

NC = 2
NS = 16
NW = NC * NS

CH = 128
E_PAD = 327680
E_ROWS = E_PAD // CH
RPW = E_ROWS // NW
N_PAD = 10240
ROWS_PER_SUB = N_PAD // NS
CNT_L = 128


def _sc_sums(emb_hbm, col2d, row2d, zacc_hbm, sums_out,
             cidx_v, ridx_v, rows_v, acc_sh, sem):
    cid = lax.axis_index("c")
    sid = lax.axis_index("s")
    wid = cid * NS + sid

    nb = sid * ROWS_PER_SUB
    pltpu.sync_copy(zacc_hbm, acc_sh.at[pl.ds(nb, ROWS_PER_SUB)])
    eb = wid * RPW
    pltpu.sync_copy(col2d.at[pl.ds(eb, RPW)], cidx_v)
    pltpu.sync_copy(row2d.at[pl.ds(eb, RPW)], ridx_v)
    plsc.subcore_barrier()

    def step(j, carry):
        pltpu.async_copy(emb_hbm.at[cidx_v.at[j]], rows_v, sem).wait()
        pltpu.sync_copy(rows_v, acc_sh.at[ridx_v.at[j]], add=True)
        return carry

    lax.fori_loop(0, RPW, step, 0)
    plsc.subcore_barrier()

    pltpu.sync_copy(acc_sh.at[pl.ds(nb, ROWS_PER_SUB)],
                    sums_out.at[cid, pl.ds(nb, ROWS_PER_SUB)])


def _sc_counts(row2d, zcnt_hbm, ones_hbm, cnts_out,
               ridx_v, ones_v, cnt_sh, sem):
    cid = lax.axis_index("c")
    sid = lax.axis_index("s")
    wid = cid * NS + sid

    nb = sid * ROWS_PER_SUB
    pltpu.sync_copy(zcnt_hbm, cnt_sh.at[pl.ds(nb, ROWS_PER_SUB)])
    eb = wid * RPW
    pltpu.sync_copy(row2d.at[pl.ds(eb, RPW)], ridx_v)
    pltpu.sync_copy(ones_hbm, ones_v)
    plsc.subcore_barrier()

    def step(j, carry):
        pltpu.sync_copy(ones_v, cnt_sh.at[ridx_v.at[j]], add=True)
        return carry

    lax.fori_loop(0, RPW, step, 0)
    plsc.subcore_barrier()

    pltpu.sync_copy(cnt_sh.at[pl.ds(nb, ROWS_PER_SUB)],
                    cnts_out.at[cid, pl.ds(nb, ROWS_PER_SUB)])


def _combine_body(sums_ref, cnts_ref, self_ref, out_ref):
    s = sums_ref[0] + sums_ref[1]
    c = cnts_ref[0, :, 0:1] + cnts_ref[1, :, 0:1]
    f1 = s / jnp.maximum(c, 1.0)
    out_ref[:, :D_FEAT] = f1
    out_ref[:, D_FEAT:] = self_ref[:] - f1


def kernel(embedding, edge_index, self_feats):
    npad = E_PAD - N_EDGES
    pad_i = jnp.arange(npad, dtype=jnp.int32)
    row = jnp.concatenate(
        [edge_index[0].astype(jnp.int32),
         N_NODES + pad_i % (N_PAD - N_NODES)]).reshape(E_ROWS, CH)
    col = jnp.concatenate(
        [edge_index[1].astype(jnp.int32),
         (pad_i * 37) % N_NODES]).reshape(E_ROWS, CH)
    zacc = jnp.zeros((ROWS_PER_SUB, D_FEAT), jnp.float32)
    zcnt = jnp.zeros((ROWS_PER_SUB, CNT_L), jnp.float32)
    ones = jnp.ones((CH, CNT_L), jnp.float32)

    mesh = plsc.VectorSubcoreMesh(core_axis_name="c", subcore_axis_name="s",
                                  num_cores=NC, num_subcores=NS)
    sums_p = pl.kernel(
        _sc_sums,
        out_type=jax.ShapeDtypeStruct((NC, N_PAD, D_FEAT), jnp.float32),
        mesh=mesh,
        scratch_types=[
            pltpu.VMEM((RPW, CH), jnp.int32),
            pltpu.VMEM((RPW, CH), jnp.int32),
            pltpu.VMEM((CH, D_FEAT), jnp.float32),
            pltpu.VMEM_SHARED((N_PAD, D_FEAT), jnp.float32),
            pltpu.SemaphoreType.DMA,
        ],
    )(embedding, col, row, zacc)

    cnts_p = pl.kernel(
        _sc_counts,
        out_type=jax.ShapeDtypeStruct((NC, N_PAD, CNT_L), jnp.float32),
        mesh=mesh,
        scratch_types=[
            pltpu.VMEM((RPW, CH), jnp.int32),
            pltpu.VMEM((CH, CNT_L), jnp.float32),
            pltpu.VMEM_SHARED((N_PAD, CNT_L), jnp.float32),
            pltpu.SemaphoreType.DMA,
        ],
    )(row, zcnt, ones)

    blk = 2000
    out = pl.pallas_call(
        _combine_body,
        grid=(N_NODES // blk,),
        in_specs=[
            pl.BlockSpec((NC, blk, D_FEAT), lambda i: (0, i, 0)),
            pl.BlockSpec((NC, blk, CNT_L), lambda i: (0, i, 0)),
            pl.BlockSpec((blk, D_FEAT), lambda i: (i, 0)),
        ],
        out_specs=pl.BlockSpec((blk, 2 * D_FEAT), lambda i: (i, 0)),
        out_shape=jax.ShapeDtypeStruct((N_NODES, 2 * D_FEAT), jnp.float32),
    )(sums_p, cnts_p, self_feats)
    return out
